# Initial kernel scaffold; baseline (speedup 1.0000x reference)
import jax
import jax.numpy as jnp
from jax import lax
from jax.experimental import pallas as pl
from jax.experimental.pallas import tpu as pltpu

_NX = 64
_NY = 64
_N = _NX * _NY
_NT = 512
_H = 1.0 / 16.0
_DT = 1.0 / 64.0
_NC = 4
_CG_ITERS = 20
_INVH2 = 1.0 / (_H * _H)


def _dr_kernel(u_ref, gamma_ref, k1t_ref, k2t_ref, k3t_ref, out_ref):
    U = u_ref[:]
    gamma = gamma_ref[:]
    K1t = k1t_ref[:]
    K2t = k2t_ref[:]
    K3t = k3t_ref[:]

    x_idx = lax.broadcasted_iota(jnp.int32, (1, _N), 1) % _NX
    y_idx = lax.broadcasted_iota(jnp.int32, (1, _N), 1) // _NX
    mL = x_idx == 0
    mR = x_idx == _NX - 1
    mU = y_idx == 0
    mD = y_idx == _NY - 1
    deg = 4.0 - (mL.astype(jnp.float32) + mR.astype(jnp.float32)
                 + mU.astype(jnp.float32) + mD.astype(jnp.float32))
    a_diag = deg * _INVH2 + gamma

    def matvec(p):
        nL = jnp.where(mL, 0.0, pltpu.roll(p, 1, 1))
        nR = jnp.where(mR, 0.0, pltpu.roll(p, -1, 1))
        nU = jnp.where(mU, 0.0, pltpu.roll(p, _NX, 1))
        nD = jnp.where(mD, 0.0, pltpu.roll(p, -_NX, 1))
        return a_diag * p - _INVH2 * ((nL + nR) + (nU + nD))

    def cg(b):
        rs0 = jnp.sum(b * b, axis=1, keepdims=True)

        def body(_, carry):
            x, r, p, rs = carry
            Ap = matvec(p)
            denom = jnp.sum(p * Ap, axis=1, keepdims=True)
            alpha = rs / (denom + 1e-30)
            x = x + alpha * p
            r = r - alpha * Ap
            rs_new = jnp.sum(r * r, axis=1, keepdims=True)
            beta = rs_new / (rs + 1e-30)
            p = r + beta * p
            return (x, r, p, rs_new)

        x, _, _, _ = lax.fori_loop(
            0, _CG_ITERS, body, (jnp.zeros_like(b), b, b, rs0))
        return x

    def step(_, U):
        Ud = cg(gamma * U)
        h = jnp.tanh(jnp.dot(K1t, Ud, preferred_element_type=jnp.float32))
        h = jnp.tanh(jnp.dot(K2t, h, preferred_element_type=jnp.float32))
        h = jnp.tanh(jnp.dot(K3t, h, preferred_element_type=jnp.float32))
        return Ud + _DT * h

    out_ref[:] = lax.fori_loop(0, _NT, step, U)


@jax.jit
def _run(U0, scale, K1, K2, K3):
    gamma = (1.0 / (scale * _DT)).reshape(_NC, 1)
    out = pl.pallas_call(
        _dr_kernel,
        out_shape=jax.ShapeDtypeStruct((_NC, _N), jnp.float32),
    )(U0.T, gamma, K1.T, K2.T, K3.T)
    return out.T


def kernel(U0, scale, K1, K2, K3):
    return _run(U0, scale, K1, K2, K3)


# speedup vs baseline: 3.4033x; 3.4033x over previous
import jax
import jax.numpy as jnp
from jax import lax
from jax.experimental import pallas as pl
from jax.experimental.pallas import tpu as pltpu

_NX = 64
_NY = 64
_N = _NX * _NY
_NT = 512
_H = 1.0 / 16.0
_DT = 1.0 / 64.0
_NC = 4
_CG_ITERS = 20
_INVH2 = 1.0 / (_H * _H)


def _dr_kernel(u_ref, gamma_ref, k1t_ref, k2t_ref, k3t_ref, out_ref):
    U = u_ref[:]
    gamma = gamma_ref[:]
    K1t = k1t_ref[:]
    K2t = k2t_ref[:]
    K3t = k3t_ref[:]

    x_idx = lax.broadcasted_iota(jnp.int32, (1, _N), 1) % _NX
    y_idx = lax.broadcasted_iota(jnp.int32, (1, _N), 1) // _NX
    mL = x_idx == 0
    mR = x_idx == _NX - 1
    mU = y_idx == 0
    mD = y_idx == _NY - 1
    deg = 4.0 - (mL.astype(jnp.float32) + mR.astype(jnp.float32)
                 + mU.astype(jnp.float32) + mD.astype(jnp.float32))
    a_diag = deg * _INVH2 + gamma

    def matvec(p):
        nL = jnp.where(mL, 0.0, pltpu.roll(p, 1, 1))
        nR = jnp.where(mR, 0.0, pltpu.roll(p, _N - 1, 1))
        nU = jnp.where(mU, 0.0, pltpu.roll(p, _NX, 1))
        nD = jnp.where(mD, 0.0, pltpu.roll(p, _N - _NX, 1))
        return a_diag * p - _INVH2 * ((nL + nR) + (nU + nD))

    def cg(b):
        rs0 = jnp.sum(b * b, axis=1, keepdims=True)

        def body(_, carry):
            x, r, p, rs = carry
            Ap = matvec(p)
            denom = jnp.sum(p * Ap, axis=1, keepdims=True)
            alpha = rs / (denom + 1e-30)
            x = x + alpha * p
            r = r - alpha * Ap
            rs_new = jnp.sum(r * r, axis=1, keepdims=True)
            beta = rs_new / (rs + 1e-30)
            p = r + beta * p
            return (x, r, p, rs_new)

        x, _, _, _ = lax.fori_loop(
            0, _CG_ITERS, body, (jnp.zeros_like(b), b, b, rs0))
        return x

    def step(_, U):
        Ud = cg(gamma * U)
        h = jnp.tanh(jnp.dot(K1t, Ud, preferred_element_type=jnp.float32))
        h = jnp.tanh(jnp.dot(K2t, h, preferred_element_type=jnp.float32))
        h = jnp.tanh(jnp.dot(K3t, h, preferred_element_type=jnp.float32))
        return Ud + _DT * h

    out_ref[:] = lax.fori_loop(0, _NT, step, U)


@jax.jit
def _run(U0, scale, K1, K2, K3):
    gamma = (1.0 / (scale * _DT)).reshape(_NC, 1)
    out = pl.pallas_call(
        _dr_kernel,
        out_shape=jax.ShapeDtypeStruct((_NC, _N), jnp.float32),
    )(U0.T, gamma, K1.T, K2.T, K3.T)
    return out.T


def kernel(U0, scale, K1, K2, K3):
    return _run(U0, scale, K1, K2, K3)


# speedup vs baseline: 3.4803x; 1.0226x over previous
import jax
import jax.numpy as jnp
from jax import lax
from jax.experimental import pallas as pl
from jax.experimental.pallas import tpu as pltpu

_NX = 64
_NY = 64
_N = _NX * _NY
_NT = 512
_H = 1.0 / 16.0
_DT = 1.0 / 64.0
_NC = 4
_CG_ITERS = 20
_INVH2 = 1.0 / (_H * _H)


def _dr_kernel(u_ref, gamma_ref, k1t_ref, k2t_ref, k3t_ref, out_ref):
    U = u_ref[:]
    gamma = gamma_ref[:]
    K1t = k1t_ref[:]
    K2t = k2t_ref[:]
    K3t = k3t_ref[:]

    x_idx = lax.broadcasted_iota(jnp.int32, (1, _N), 1) % _NX
    y_idx = lax.broadcasted_iota(jnp.int32, (1, _N), 1) // _NX
    mL = x_idx == 0
    mR = x_idx == _NX - 1
    mU = y_idx == 0
    mD = y_idx == _NY - 1
    deg = 4.0 - (mL.astype(jnp.float32) + mR.astype(jnp.float32)
                 + mU.astype(jnp.float32) + mD.astype(jnp.float32))
    a_diag = deg * _INVH2 + gamma

    def matvec(p):
        nL = jnp.where(mL, 0.0, pltpu.roll(p, 1, 1))
        nR = jnp.where(mR, 0.0, pltpu.roll(p, _N - 1, 1))
        nU = jnp.where(mU, 0.0, pltpu.roll(p, _NX, 1))
        nD = jnp.where(mD, 0.0, pltpu.roll(p, _N - _NX, 1))
        return a_diag * p - _INVH2 * ((nL + nR) + (nU + nD))

    def cg(b):
        rs0 = jnp.sum(b * b, axis=1, keepdims=True)

        def body(_, carry):
            x, r, p, Ap, rs = carry
            denom = jnp.sum(p * Ap, axis=1, keepdims=True)
            alpha = rs / (denom + 1e-30)
            x = x + alpha * p
            r = r - alpha * Ap
            w = matvec(r)
            rs_new = jnp.sum(r * r, axis=1, keepdims=True)
            beta = rs_new / (rs + 1e-30)
            p = r + beta * p
            Ap = w + beta * Ap
            return (x, r, p, Ap, rs_new)

        x, _, _, _, _ = lax.fori_loop(
            0, _CG_ITERS, body, (jnp.zeros_like(b), b, b, matvec(b), rs0))
        return x

    def step(_, U):
        Ud = cg(gamma * U)
        h = jnp.tanh(jnp.dot(K1t, Ud, preferred_element_type=jnp.float32))
        h = jnp.tanh(jnp.dot(K2t, h, preferred_element_type=jnp.float32))
        h = jnp.tanh(jnp.dot(K3t, h, preferred_element_type=jnp.float32))
        return Ud + _DT * h

    out_ref[:] = lax.fori_loop(0, _NT, step, U)


@jax.jit
def _run(U0, scale, K1, K2, K3):
    gamma = (1.0 / (scale * _DT)).reshape(_NC, 1)
    out = pl.pallas_call(
        _dr_kernel,
        out_shape=jax.ShapeDtypeStruct((_NC, _N), jnp.float32),
    )(U0.T, gamma, K1.T, K2.T, K3.T)
    return out.T


def kernel(U0, scale, K1, K2, K3):
    return _run(U0, scale, K1, K2, K3)


# speedup vs baseline: 4.9876x; 1.4331x over previous
import jax
import jax.numpy as jnp
from jax import lax
from jax.experimental import pallas as pl
from jax.experimental.pallas import tpu as pltpu

_NX = 64
_NY = 64
_N = _NX * _NY
_NT = 512
_H = 1.0 / 16.0
_DT = 1.0 / 64.0
_NC = 4
_CG_ITERS = 20
_INVH2 = 1.0 / (_H * _H)
_R = 32
_L = 512


def _dr_kernel(u_ref, gamma_ref, ad_ref, w1_ref, w2_ref, w3_ref, out_ref):
    G = gamma_ref[:]
    AD = ad_ref[:]
    W1 = w1_ref[:]
    W2 = w2_ref[:]
    W3 = w3_ref[:]

    lane = lax.broadcasted_iota(jnp.int32, (1, _L), 1)
    row = lax.broadcasted_iota(jnp.int32, (8, 1), 0)
    mx0 = lane % _NX == 0
    mx63 = lane % _NX == _NX - 1
    mfc = lane < _NX
    mlc = lane >= _L - _NX
    my0 = (row == 0) & mfc
    my63 = (row == 7) & mlc

    def matvec(p, ad):
        uL = jnp.where(mx0, 0.0, pltpu.roll(p, 1, 1))
        uR = jnp.where(mx63, 0.0, pltpu.roll(p, _L - 1, 1))
        a = pltpu.roll(p, _NX, 1)
        uU = jnp.where(my0, 0.0, jnp.where(mfc, pltpu.roll(a, 1, 0), a))
        c = pltpu.roll(p, _L - _NX, 1)
        uD = jnp.where(my63, 0.0, jnp.where(mlc, pltpu.roll(c, 7, 0), c))
        return ad * p - _INVH2 * ((uL + uR) + (uU + uD))

    def psum(z):
        s = jnp.sum(z, axis=1, keepdims=True)
        s = s + pltpu.roll(s, 4, 0)
        s = s + pltpu.roll(s, 2, 0)
        s = s + pltpu.roll(s, 1, 0)
        return s

    ads = [AD[8 * c:8 * c + 8, :] for c in range(_NC)]
    gs = [G[8 * c:8 * c + 8, :] for c in range(_NC)]

    def step(_, U):
        b = [gs[c] * U[8 * c:8 * c + 8, :] for c in range(_NC)]
        x = [jnp.zeros_like(bc) for bc in b]
        r = list(b)
        p = list(b)
        Ap = [matvec(b[c], ads[c]) for c in range(_NC)]
        rs = [psum(b[c] * b[c]) for c in range(_NC)]
        den = [psum(p[c] * Ap[c]) for c in range(_NC)]
        for it in range(_CG_ITERS):
            for c in range(_NC):
                alpha = rs[c] / (den[c] + 1e-30)
                x[c] = x[c] + alpha * p[c]
                if it == _CG_ITERS - 1:
                    continue
                invrs = 1.0 / (rs[c] + 1e-30)
                r[c] = r[c] - alpha * Ap[c]
                w = matvec(r[c], ads[c])
                rsn = psum(r[c] * r[c])
                rw = psum(r[c] * w)
                rap = psum(r[c] * Ap[c])
                beta = rsn * invrs
                den[c] = rw + beta * (2.0 * rap + beta * den[c])
                p[c] = r[c] + beta * p[c]
                Ap[c] = w + beta * Ap[c]
                rs[c] = rsn
        Ud = jnp.concatenate(x, axis=0)
        h = jnp.tanh(jnp.dot(W1, Ud, preferred_element_type=jnp.float32))
        h = jnp.tanh(jnp.dot(W2, h, preferred_element_type=jnp.float32))
        h = jnp.tanh(jnp.dot(W3, h, preferred_element_type=jnp.float32))
        return Ud + _DT * h

    out_ref[:] = lax.fori_loop(0, _NT, step, u_ref[:], unroll=2)


@jax.jit
def _run(U0, scale, K1, K2, K3):
    gamma = jnp.repeat(1.0 / (scale * _DT), 8).reshape(_R, 1)
    yi = jnp.arange(_NY)
    degy = 2.0 - (yi == 0) - (yi == _NY - 1)
    xi = jnp.arange(_NX)
    degx = 2.0 - (xi == 0) - (xi == _NX - 1)
    deg = (degy[:, None] + degx[None, :]).astype(jnp.float32)
    deg8 = deg.reshape(8, _L)
    a_diag = jnp.tile(deg8, (_NC, 1)) * _INVH2 + gamma
    X = U0.reshape(8, 8, _NX, _NC).transpose(3, 0, 1, 2).reshape(_R, _L)
    eye8 = jnp.eye(8, dtype=jnp.float32)
    W1 = jnp.kron(K1.T, eye8)
    W2 = jnp.kron(K2.T, eye8)
    W3 = jnp.kron(K3.T, eye8)
    out = pl.pallas_call(
        _dr_kernel,
        out_shape=jax.ShapeDtypeStruct((_R, _L), jnp.float32),
    )(X, gamma, a_diag, W1, W2, W3)
    return out.reshape(_NC, 8, 8, _NX).transpose(1, 2, 3, 0).reshape(_N, _NC)


def kernel(U0, scale, K1, K2, K3):
    return _run(U0, scale, K1, K2, K3)
